# baseline (speedup 1.0000x reference)
import functools

import jax
import jax.numpy as jnp
from jax import lax
from jax.experimental import pallas as pl
from jax.experimental.pallas import tpu as pltpu
from jax.experimental.pallas import tpu_sc as plsc

N = 10000
E = 320000
F = 128
H = 4
NTILES = 16
EALL = E + N
C = 128
NCH = 2592
EALL_PAD = NCH * C
CPW_A = NCH // 32
CPT_B = NCH // NTILES
DSTRIDE = 624
DWIN = 640


def _tc_proj(features, Wcat, bcat, Acat):
    BN = 1000

    def body(x_ref, w_ref, b_ref, a_ref, h_ref, s_ref):
        h = jnp.dot(x_ref[...], w_ref[...],
                    preferred_element_type=jnp.float32) + b_ref[...]
        for k in range(H):
            h_ref[k] = h[:, k * F:(k + 1) * F]
        s_ref[...] = jnp.dot(h, a_ref[...], preferred_element_type=jnp.float32)

    return pl.pallas_call(
        body,
        grid=(N // BN,),
        in_specs=[
            pl.BlockSpec((BN, F), lambda i: (i, 0)),
            pl.BlockSpec((F, H * F), lambda i: (0, 0)),
            pl.BlockSpec((1, H * F), lambda i: (0, 0)),
            pl.BlockSpec((H * F, 128), lambda i: (0, 0)),
        ],
        out_specs=[
            pl.BlockSpec((H, BN, F), lambda i: (0, i, 0)),
            pl.BlockSpec((BN, 128), lambda i: (i, 0)),
        ],
        out_shape=[
            jax.ShapeDtypeStruct((H, N, F), jnp.float32),
            jax.ShapeDtypeStruct((N, 128), jnp.float32),
        ],
    )(features, Wcat, bcat, Acat)


def _sc_weights(sdst, ssrc, src_all, dst_all):
    mesh = plsc.VectorSubcoreMesh(core_axis_name="c", subcore_axis_name="s")

    @functools.partial(
        pl.kernel, mesh=mesh,
        out_type=jax.ShapeDtypeStruct((NCH, H, C), jnp.float32),
        compiler_params=pltpu.CompilerParams(needs_layout_passes=False),
        scratch_types=[
            pltpu.VMEM((H * N,), jnp.float32),
            pltpu.VMEM((H * N,), jnp.float32),
            pltpu.VMEM((C,), jnp.int32),
            pltpu.VMEM((C,), jnp.int32),
            pltpu.VMEM((C,), jnp.int32),
            pltpu.VMEM((C,), jnp.int32),
            pltpu.VMEM((H, C), jnp.float32),
            pltpu.VMEM((H, C), jnp.float32),
            pltpu.SemaphoreType.DMA,
            pltpu.SemaphoreType.DMA,
            pltpu.SemaphoreType.DMA,
            pltpu.SemaphoreType.DMA,
        ],
    )
    def ka(sdst_hbm, ssrc_hbm, src_hbm, dst_hbm, w_hbm,
           sd4, ss4, src_a, src_b, dst_a, dst_b, w_a, w_b,
           isem_a, isem_b, wsem_a, wsem_b):
        c = lax.axis_index("c")
        s = lax.axis_index("s")
        wid = s * 2 + c
        g0 = wid * CPW_A
        pltpu.sync_copy(sdst_hbm, sd4)
        pltpu.sync_copy(ssrc_hbm, ss4)
        iot = jnp.arange(16, dtype=jnp.int32)

        def idx_copies(j, sbuf, dbuf, sem):
            o = (g0 + j) * C
            return (pltpu.make_async_copy(src_hbm.at[pl.ds(o, C)], sbuf, sem),
                    pltpu.make_async_copy(dst_hbm.at[pl.ds(o, C)], dbuf, sem))

        def w_copy(j, wbuf, sem):
            return pltpu.make_async_copy(wbuf, w_hbm.at[g0 + j], sem)

        def compute(j, sbuf, dbuf, wbuf):
            o = (g0 + j) * C
            for i in range(C // 16):
                sl = pl.ds(i * 16, 16)
                s16 = sbuf[sl]
                d16 = dbuf[sl]
                valid = (o + i * 16 + iot) < EALL
                for head in range(H):
                    e = (plsc.load_gather(sd4, [d16 + head * N])
                         + plsc.load_gather(ss4, [s16 + head * N]))
                    e = jnp.where(e > 0, e, e * 0.2)
                    wbuf[head, sl] = jnp.where(valid, jnp.exp(e), 0.0)

        for cp in idx_copies(0, src_a, dst_a, isem_a):
            cp.start()

        bufs = ((src_a, dst_a, w_a, isem_a, wsem_a),
                (src_b, dst_b, w_b, isem_b, wsem_b))

        def phase(jj, j, p):
            sbuf, dbuf, wbuf, isem, wsem = bufs[p]
            nsbuf, ndbuf = bufs[1 - p][0], bufs[1 - p][1]
            nisem = bufs[1 - p][3]
            for cp in idx_copies(j + 1, nsbuf, ndbuf, nisem):
                cp.start()
            for cp in idx_copies(j, sbuf, dbuf, isem):
                cp.wait()

            @pl.when(jj >= 1)
            def _():
                w_copy(j - 2, wbuf, wsem).wait()
            compute(j, sbuf, dbuf, wbuf)
            w_copy(j, wbuf, wsem).start()

        def loop(jj, _):
            phase(jj, 2 * jj, 0)
            phase(jj, 2 * jj + 1, 1)
            return 0
        lax.fori_loop(0, (CPW_A - 1) // 2, loop, 0)

        jt = CPW_A - 1
        for cp in idx_copies(jt, src_a, dst_a, isem_a):
            cp.wait()
        w_copy(jt - 2, w_a, wsem_a).wait()
        compute(jt, src_a, dst_a, w_a)
        w_copy(jt, w_a, wsem_a).start()
        w_copy(jt - 1, w_b, wsem_b).wait()
        w_copy(jt, w_a, wsem_a).wait()

    return ka(sdst, ssrc, src_all, dst_all)


def _sc_aggregate(h_flat, w_all, src_all, dst_all):
    mesh = plsc.VectorSubcoreMesh(core_axis_name="c", subcore_axis_name="s")

    @functools.partial(
        pl.kernel, mesh=mesh,
        out_type=jax.ShapeDtypeStruct((N, H, F), jnp.float32),
        compiler_params=pltpu.CompilerParams(needs_layout_passes=False),
        scratch_types=[
            pltpu.VMEM((C, F), jnp.float32),
            pltpu.VMEM((C, F), jnp.float32),
            pltpu.VMEM((C, F), jnp.float32),
            pltpu.VMEM((C,), jnp.int32),
            pltpu.VMEM((C,), jnp.int32),
            pltpu.VMEM((C,), jnp.int32),
            pltpu.VMEM((C,), jnp.int32),
            pltpu.VMEM((C,), jnp.int32),
            pltpu.VMEM((C,), jnp.int32),
            pltpu.VMEM((C,), jnp.float32),
            pltpu.VMEM((C,), jnp.float32),
            pltpu.VMEM((C,), jnp.float32),
            pltpu.VMEM_SHARED((N, F), jnp.float32),
            pltpu.VMEM_SHARED((N,), jnp.float32),
            pltpu.SemaphoreType.DMA,
            pltpu.SemaphoreType.DMA,
            pltpu.SemaphoreType.DMA,
            pltpu.SemaphoreType.DMA,
            pltpu.SemaphoreType.DMA,
            pltpu.SemaphoreType.DMA,
            pltpu.SemaphoreType.DMA,
            pltpu.SemaphoreType.DMA,
            pltpu.SemaphoreType.DMA,
        ],
    )
    def kb(h_hbm, w_hbm, src_hbm, dst_hbm, out_hbm,
           rows_0, rows_1, rows_2, src_0, src_1, src_2,
           dst_0, dst_1, dst_2, w_0, w_1, w_2,
           acc_sh, den_sh,
           isem_0, isem_1, isem_2, gsem_0, gsem_1, gsem_2,
           ssem_0, ssem_1, ssem_2):
        c = lax.axis_index("c")
        s = lax.axis_index("s")
        zeros16 = jnp.zeros((16,), jnp.float32)
        base_row = s * DSTRIDE
        g0 = s * CPT_B
        bufs = ((rows_0, src_0, dst_0, w_0, isem_0, gsem_0, ssem_0),
                (rows_1, src_1, dst_1, w_1, isem_1, gsem_1, ssem_1),
                (rows_2, src_2, dst_2, w_2, isem_2, gsem_2, ssem_2))
        NBLK = DWIN // C

        for kk in range(2):
            head = 2 * c + kk

            def zacc(r, _):
                for i2 in range(F // 16):
                    rows_0[r, pl.ds(i2 * 16, 16)] = zeros16
                return 0
            lax.fori_loop(0, C, zacc, 0)
            for i2 in range(F // 16):
                rows_1[0, pl.ds(i2 * 16, 16)] = zeros16
            for zb in range(NBLK):
                pltpu.sync_copy(rows_0,
                                acc_sh.at[pl.ds(base_row + zb * C, C)])
                pltpu.sync_copy(rows_1.at[0],
                                den_sh.at[pl.ds(base_row + zb * C, C)])
            plsc.subcore_barrier()

            def idx_copies(j, p):
                rows, sbuf, dbuf, wbuf, isem = bufs[p][:5]
                o = (g0 + j) * C
                return (
                    pltpu.make_async_copy(src_hbm.at[pl.ds(o, C)], sbuf, isem),
                    pltpu.make_async_copy(dst_hbm.at[pl.ds(o, C)], dbuf, isem),
                    pltpu.make_async_copy(
                        w_hbm.at[pl.ds(((g0 + j) * H + head) * C, C)],
                        wbuf, isem),
                )

            def gather_copy(p):
                rows, sbuf = bufs[p][0], bufs[p][1]
                return pltpu.make_async_copy(h_hbm.at[sbuf], rows, bufs[p][5])

            def scatter_copies(p):
                rows, sbuf, dbuf, wbuf = bufs[p][:4]
                ssem = bufs[p][6]
                return (pltpu.make_async_copy(rows, acc_sh.at[dbuf], ssem),
                        pltpu.make_async_copy(wbuf, den_sh.at[dbuf], ssem))

            def prefetch(j, p):
                for cp in idx_copies(j, p):
                    cp.start()
                for cp in idx_copies(j, p):
                    cp.wait()
                sbuf = bufs[p][1]
                for i in range(C // 16):
                    sl = pl.ds(i * 16, 16)
                    sbuf[sl] = sbuf[sl] + head * N
                gather_copy(p).start()

            def process(p):
                rows, wbuf = bufs[p][0], bufs[p][3]
                gather_copy(p).wait()

                def body(q, _):
                    wv = wbuf[pl.ds(q * 16, 16)]
                    for ri in range(16):
                        a = wv[ri]
                        r = q * 16 + ri
                        for i2 in range(F // 16):
                            sl2 = pl.ds(i2 * 16, 16)
                            rows[r, sl2] = rows[r, sl2] * a
                    return 0
                lax.fori_loop(0, C // 16, body, 0)
                r_cp, w_cp = scatter_copies(p)
                r_cp.start(add=True)
                w_cp.start(add=True)

            def wait_scatter(p):
                for cp in scatter_copies(p):
                    cp.wait()

            prefetch(0, 0)
            prefetch(1, 1)

            def loop(jj, _):
                j0 = 3 * jj
                process(0)

                @pl.when(jj >= 1)
                def _():
                    wait_scatter(2)
                prefetch(j0 + 2, 2)

                process(1)
                wait_scatter(0)

                @pl.when(jj < CPT_B // 3 - 1)
                def _():
                    prefetch(j0 + 3, 0)

                process(2)
                wait_scatter(1)

                @pl.when(jj < CPT_B // 3 - 1)
                def _():
                    prefetch(j0 + 4, 1)
                return 0
            lax.fori_loop(0, CPT_B // 3, loop, 0)
            wait_scatter(2)
            plsc.subcore_barrier()

            for blk in range(NBLK):
                r0 = base_row + blk * C
                pltpu.sync_copy(acc_sh.at[pl.ds(r0, C)], rows_0)
                pltpu.sync_copy(den_sh.at[pl.ds(r0, C)], rows_1.at[0])

                def div(q, _):
                    iv = 1.0 / rows_1[0, pl.ds(q * 16, 16)]
                    for ri in range(16):
                        a = iv[ri]
                        r = q * 16 + ri
                        for i2 in range(F // 16):
                            sl2 = pl.ds(i2 * 16, 16)
                            rows_0[r, sl2] = rows_0[r, sl2] * a
                    return 0
                lax.fori_loop(0, C // 16, div, 0)
                pltpu.sync_copy(rows_0, out_hbm.at[pl.ds(r0, C), head])
            plsc.subcore_barrier()

    return kb(h_flat, w_all, src_all, dst_all)


def kernel(features, edge_index, W, b, a_w, a_b):
    Wcat = jnp.transpose(W, (2, 0, 1)).reshape(F, H * F)
    bcat = b.reshape(1, H * F)
    Acat = jnp.zeros((H * F, 128), jnp.float32)
    for k in range(H):
        Acat = Acat.at[k * F:(k + 1) * F, k].set(a_w[k, :F])
        Acat = Acat.at[k * F:(k + 1) * F, H + k].set(a_w[k, F:])

    h4, sdss = _tc_proj(features, Wcat, bcat, Acat)

    sd = sdss[:, :H].T + a_b[:, None]
    ss = sdss[:, H:2 * H].T
    sdst = sd.reshape(-1)
    ssrc = ss.reshape(-1)

    loop = jnp.arange(N, dtype=jnp.int32)
    pad_e = EALL_PAD - EALL
    src_all = jnp.concatenate(
        [edge_index[0], loop, jnp.zeros((pad_e,), jnp.int32)])
    dst_all = jnp.concatenate(
        [edge_index[1], loop, jnp.zeros((pad_e,), jnp.int32)])

    w_all = _sc_weights(sdst, ssrc, src_all, dst_all)
    out = _sc_aggregate(h4.reshape(H * N, F), w_all.reshape(-1),
                        src_all, dst_all)
    return out.reshape(N, H * F)
